# baseline (speedup 1.0000x reference)
import jax
import jax.numpy as jnp
from jax import lax
from jax.experimental import pallas as pl
from jax.experimental.pallas import tpu as pltpu
from jax.experimental.pallas import tpu_sc as plsc

B, S, D_RHO, D_C, HID = 16, 2048, 256, 128, 512
GRID = 17
K_PAD = 384
NC, L = 2, 16
SH = S // NC
NB = B * K_PAD


def _sc_hist_body(gidx_hbm, zeros_hbm, out_hbm, idxv, onesv, shared):
    c = lax.axis_index("c")
    s = lax.axis_index("s")
    pltpu.sync_copy(gidx_hbm.at[s, c], idxv)
    for i in range(8):
        onesv[pl.ds(i * L, L)] = jnp.ones((L,), jnp.float32)

    @pl.when(s == 0)
    def _zero():
        pltpu.sync_copy(zeros_hbm, shared)
    plsc.subcore_barrier()

    for j in range(SH // 128):
        pltpu.sync_copy(onesv, shared.at[idxv.at[j]], add=True)
    plsc.subcore_barrier()

    @pl.when(s == 0)
    def _flush():
        pltpu.sync_copy(shared, out_hbm.at[c])


def _sc_histogram(gidx, zeros):
    mesh = plsc.VectorSubcoreMesh(core_axis_name="c", subcore_axis_name="s")
    return pl.kernel(
        _sc_hist_body,
        mesh=mesh,
        out_type=jax.ShapeDtypeStruct((NC, NB), jnp.float32),
        scratch_types=[
            pltpu.VMEM((SH // 128, 128), jnp.int32),
            pltpu.VMEM((128,), jnp.float32),
            pltpu.VMEM_SHARED((NB,), jnp.float32),
        ],
    )(gidx, zeros)


def _mlp_bias(hist, cflat_ref, W1_ref, b1_ref, W2_ref, b2_ref,
              Wf1c_ref, bf1_ref):
    mean_c = jnp.dot(hist * (1.0 / S), cflat_ref[...])
    pre = jnp.dot(mean_c, W1_ref[...]) + b1_ref[...]
    scale, alpha = 1.0507009873554805, 1.6732632423543772
    h2 = scale * jnp.where(pre > 0, pre, alpha * (jnp.exp(pre) - 1.0))
    c_tr = jnp.dot(h2, W2_ref[...]) + b2_ref[...]
    return jnp.dot(c_tr, Wf1c_ref[...]) + bf1_ref[...]


def _hist_row(ph_ref, r):
    return (ph_ref[0, pl.ds(r, 1), :] + ph_ref[1, pl.ds(r, 1), :])


def _tc_body(ph_ref, rho_ref, wcol_ref, cflat_ref, W1_ref, b1_ref, W2_ref,
             b2_ref, Wf1r_ref, Wf1c_ref, bf1_ref, W2b_ref,
             outm_ref, outv_ref, bias_sc, lmv_sc):
    b = pl.program_id(0)

    @pl.when(b == 0)
    def _prologue():
        bias_sc[pl.ds(0, 1), :] = _mlp_bias(
            _hist_row(ph_ref, 0), cflat_ref, W1_ref, b1_ref, W2_ref, b2_ref,
            Wf1c_ref, bf1_ref)

    lmp = lmv_sc[pl.ds(1 - b % 2, 1)][0]
    w = wcol_ref[0]
    lm = lmp[:, 0:1]
    lv = lmp[:, 1:2]
    ma = jnp.max(lm)
    mb = jnp.max(lv)
    sa = jnp.sum(w * jnp.exp(lm - ma))
    sb = jnp.sum((w * w) * jnp.exp(lv - mb))
    outm_ref[0] = jnp.full((8, 128), ma + jnp.log(sa), dtype=jnp.float32)
    outv_ref[0] = jnp.full((8, 128), mb + jnp.log(sb), dtype=jnp.float32)

    bias = bias_sc[pl.ds(jnp.minimum(b, B - 1), 1), :]
    acc = jnp.dot(rho_ref[0].astype(jnp.bfloat16), Wf1r_ref[...],
                  preferred_element_type=jnp.float32)
    hf = jnp.maximum(acc + bias, 0.0).astype(jnp.bfloat16)
    lmv = jnp.dot(hf, W2b_ref[...],
                  preferred_element_type=jnp.float32)
    lmv_sc[pl.ds(b % 2, 1)] = lmv[None]

    bias_sc[pl.ds(jnp.minimum(b + 1, B - 1), 1), :] = _mlp_bias(
        _hist_row(ph_ref, jnp.minimum(b + 1, B - 1)), cflat_ref, W1_ref,
        b1_ref, W2_ref, b2_ref, Wf1c_ref, bf1_ref)


@jax.jit
def kernel(rho, c, w, l, roads, lon_idx, lat_idx, W1, b1, W2, b2, Wf1, bf1,
           W21, b21, W22, b22):
    cc = jnp.transpose(jnp.squeeze(c, axis=0), (1, 2, 0))
    c_flat = cc.reshape(GRID * GRID, D_C)
    c_flat = jnp.pad(c_flat, ((0, K_PAD - GRID * GRID), (0, 0)))

    idx = (lat_idx.astype(jnp.int32) * GRID + lon_idx.astype(jnp.int32))
    gidx = idx + jnp.arange(B, dtype=jnp.int32)[:, None] * K_PAD
    gidx = gidx.reshape(B, NC, SH // 128, 128)
    zeros = jnp.zeros((NB,), jnp.float32)
    w_col = w.reshape(B, S, 1)

    Wf1_r = Wf1[:D_RHO].astype(jnp.bfloat16)
    Wf1_c = Wf1[D_RHO:]
    W2b = jnp.pad(jnp.concatenate([W21, W22], axis=1),
                  ((0, 0), (0, 126))).astype(jnp.bfloat16)

    phist = _sc_histogram(gidx, zeros).reshape(NC, B, K_PAD)

    full = lambda shp: pl.BlockSpec(shp, lambda b: (0,) * len(shp))
    outm, outv = pl.pallas_call(
        _tc_body,
        grid=(B + 1,),
        in_specs=[
            full((NC, B, K_PAD)),
            pl.BlockSpec((1, S, D_RHO), lambda b: (jnp.minimum(b, B - 1), 0, 0)),
            pl.BlockSpec((1, S, 1), lambda b: (jnp.maximum(b - 1, 0), 0, 0)),
            full((K_PAD, D_C)),
            full((D_C, 256)),
            full((1, 256)),
            full((256, D_C)),
            full((1, D_C)),
            full((D_RHO, HID)),
            full((D_C, HID)),
            full((1, HID)),
            full((HID, 128)),
        ],
        out_specs=[
            pl.BlockSpec((1, 8, 128), lambda b: (jnp.maximum(b - 1, 0), 0, 0)),
            pl.BlockSpec((1, 8, 128), lambda b: (jnp.maximum(b - 1, 0), 0, 0)),
        ],
        out_shape=[
            jax.ShapeDtypeStruct((B, 8, 128), jnp.float32),
            jax.ShapeDtypeStruct((B, 8, 128), jnp.float32),
        ],
        scratch_shapes=[
            pltpu.VMEM((B, HID), jnp.float32),
            pltpu.VMEM((2, S, 128), jnp.float32),
        ],
        compiler_params=pltpu.CompilerParams(
            dimension_semantics=("arbitrary",)),
    )(phist, rho, w_col, c_flat, W1, b1.reshape(1, 256), W2,
      b2.reshape(1, D_C), Wf1_r, Wf1_c, bf1.reshape(1, HID), W2b)

    logm_agg = outm[:, 0, 0] + b21[0]
    logv_agg = outv[:, 0, 0] + b22[0]
    logl = jnp.log(l)
    return (logl - logm_agg, logl - 3.0 * logm_agg - logv_agg)


# speedup vs baseline: 1.1982x; 1.1982x over previous
import jax
import jax.numpy as jnp
from jax.experimental import pallas as pl
from jax.experimental.pallas import tpu as pltpu

B, S, D_RHO, D_C, HID = 16, 2048, 256, 128, 512
GRID = 17
K_PAD = 384


def _hist(idx_col):
    bins = jax.lax.broadcasted_iota(jnp.int32, (S, K_PAD), 1)
    onehot = (bins == idx_col).astype(jnp.float32)
    return jnp.sum(onehot, axis=0, keepdims=True)


def _mlp_bias(hist, cflat_ref, W1_ref, b1_ref, W2_ref, b2_ref,
              Wf1c_ref, bf1_ref):
    mean_c = jnp.dot(hist * (1.0 / S), cflat_ref[...])
    pre = jnp.dot(mean_c, W1_ref[...]) + b1_ref[...]
    scale, alpha = 1.0507009873554805, 1.6732632423543772
    h2 = scale * jnp.where(pre > 0, pre, alpha * (jnp.exp(pre) - 1.0))
    c_tr = jnp.dot(h2, W2_ref[...]) + b2_ref[...]
    return jnp.dot(c_tr, Wf1c_ref[...]) + bf1_ref[...]


def _tc_body(idxn_ref, idx0_ref, idx1_ref, rho_ref, wcol_ref, cflat_ref,
             W1_ref, b1_ref, W2_ref, b2_ref, Wf1r_ref, Wf1c_ref, bf1_ref,
             W2b_ref, outm_ref, outv_ref, bias_sc, lmv_sc, hist_sc):
    b = pl.program_id(0)

    @pl.when(b == 0)
    def _prologue():
        bias_sc[pl.ds(0, 1), :] = _mlp_bias(
            _hist(idx0_ref[0]), cflat_ref, W1_ref, b1_ref, W2_ref, b2_ref,
            Wf1c_ref, bf1_ref)
        hist_sc[pl.ds(1, 1)] = _hist(idx1_ref[0])[None]

    lmp = lmv_sc[pl.ds(1 - b % 2, 1)][0]
    w = wcol_ref[0]
    lm = lmp[:, 0:1]
    lv = lmp[:, 1:2]
    ma = jnp.max(lm)
    mb = jnp.max(lv)
    sa = jnp.sum(w * jnp.exp(lm - ma))
    sb = jnp.sum((w * w) * jnp.exp(lv - mb))
    outm_ref[0] = jnp.full((8, 128), ma + jnp.log(sa), dtype=jnp.float32)
    outv_ref[0] = jnp.full((8, 128), mb + jnp.log(sb), dtype=jnp.float32)

    bias = bias_sc[pl.ds(jnp.minimum(b, B - 1), 1), :]
    acc = jnp.dot(rho_ref[0].astype(jnp.bfloat16), Wf1r_ref[...],
                  preferred_element_type=jnp.float32)
    hf = jnp.maximum(acc + bias, 0.0).astype(jnp.bfloat16)
    lmv = jnp.dot(hf, W2b_ref[...],
                  preferred_element_type=jnp.float32)
    lmv_sc[pl.ds(b % 2, 1)] = lmv[None]

    histp = hist_sc[pl.ds(1 - b % 2, 1)][0]
    bias_sc[pl.ds(jnp.minimum(b + 1, B - 1), 1), :] = _mlp_bias(
        histp, cflat_ref, W1_ref, b1_ref, W2_ref, b2_ref, Wf1c_ref, bf1_ref)

    hist_sc[pl.ds(b % 2, 1)] = _hist(idxn_ref[0])[None]


@jax.jit
def kernel(rho, c, w, l, roads, lon_idx, lat_idx, W1, b1, W2, b2, Wf1, bf1,
           W21, b21, W22, b22):
    cc = jnp.transpose(jnp.squeeze(c, axis=0), (1, 2, 0))
    c_flat = cc.reshape(GRID * GRID, D_C)
    c_flat = jnp.pad(c_flat, ((0, K_PAD - GRID * GRID), (0, 0)))

    idx = (lat_idx.astype(jnp.int32) * GRID + lon_idx.astype(jnp.int32))
    idx_col = idx.reshape(B, S, 1)
    w_col = w.reshape(B, S, 1)

    Wf1_r = Wf1[:D_RHO].astype(jnp.bfloat16)
    Wf1_c = Wf1[D_RHO:]
    W2b = jnp.pad(jnp.concatenate([W21, W22], axis=1),
                  ((0, 0), (0, 126))).astype(jnp.bfloat16)

    full = lambda shp: pl.BlockSpec(shp, lambda b: (0,) * len(shp))
    outm, outv = pl.pallas_call(
        _tc_body,
        grid=(B + 1,),
        in_specs=[
            pl.BlockSpec((1, S, 1), lambda b: (jnp.minimum(b + 2, B - 1), 0, 0)),
            pl.BlockSpec((1, S, 1), lambda b: (0, 0, 0)),
            pl.BlockSpec((1, S, 1), lambda b: (1, 0, 0)),
            pl.BlockSpec((1, S, D_RHO), lambda b: (jnp.minimum(b, B - 1), 0, 0)),
            pl.BlockSpec((1, S, 1), lambda b: (jnp.maximum(b - 1, 0), 0, 0)),
            full((K_PAD, D_C)),
            full((D_C, 256)),
            full((1, 256)),
            full((256, D_C)),
            full((1, D_C)),
            full((D_RHO, HID)),
            full((D_C, HID)),
            full((1, HID)),
            full((HID, 128)),
        ],
        out_specs=[
            pl.BlockSpec((1, 8, 128), lambda b: (jnp.maximum(b - 1, 0), 0, 0)),
            pl.BlockSpec((1, 8, 128), lambda b: (jnp.maximum(b - 1, 0), 0, 0)),
        ],
        out_shape=[
            jax.ShapeDtypeStruct((B, 8, 128), jnp.float32),
            jax.ShapeDtypeStruct((B, 8, 128), jnp.float32),
        ],
        scratch_shapes=[
            pltpu.VMEM((B, HID), jnp.float32),
            pltpu.VMEM((2, S, 128), jnp.float32),
            pltpu.VMEM((2, 1, K_PAD), jnp.float32),
        ],
        compiler_params=pltpu.CompilerParams(
            dimension_semantics=("arbitrary",)),
    )(idx_col, idx_col, idx_col, rho, w_col, c_flat, W1, b1.reshape(1, 256),
      W2, b2.reshape(1, D_C), Wf1_r, Wf1_c, bf1.reshape(1, HID), W2b)

    logm_agg = outm[:, 0, 0] + b21[0]
    logv_agg = outv[:, 0, 0] + b22[0]
    logl = jnp.log(l)
    return (logl - logm_agg, logl - 3.0 * logm_agg - logv_agg)
